# Initial kernel scaffold; baseline (speedup 1.0000x reference)
import jax
import jax.numpy as jnp
from jax.experimental import pallas as pl


def kernel(node_features, edge_index, edge_attr_e3nn, node_attr_scalar_raw, W_tp_msg, W_lin_msg, W_tp_upd, W_lin_upd):
    raise NotImplementedError("write your pallas kernel here")


# speedup vs baseline: 1.5254x; 1.5254x over previous
import functools
import math

import jax
import jax.numpy as jnp
from jax import lax
from jax.experimental import pallas as pl
from jax.experimental.pallas import tpu as pltpu
from jax.experimental.pallas import tpu_sc as plsc

N, E, D, DE, H = 10000, 160000, 128, 4, 64

NW = 32
EPW = E // NW
CH = 40
NCH = EPW // CH
NP_PAD = 10240
NPT = NP_PAD // 16

@functools.cache
def _build_sc_gather():
    mesh = plsc.VectorSubcoreMesh(core_axis_name="c", subcore_axis_name="s")

    @functools.partial(
        pl.kernel,
        out_type=jax.ShapeDtypeStruct((E, D), jnp.float32),
        mesh=mesh,
        scratch_types=[
            pltpu.VMEM((NCH, CH), jnp.int32),
            pltpu.VMEM((CH, D), jnp.float32),
            pltpu.SemaphoreType.DMA,
        ],
    )
    def sc_gather(x_hbm, col_hbm, out_hbm, idx_v, rows_v, sem):
        wid = lax.axis_index("s") * 2 + lax.axis_index("c")
        pltpu.sync_copy(col_hbm.at[wid], idx_v)

        def body(j, _):
            pltpu.async_copy(x_hbm.at[idx_v.at[j]], rows_v, sem).wait()
            pltpu.sync_copy(rows_v, out_hbm.at[pl.ds(wid * EPW + j * CH, CH)])
            return ()

        lax.fori_loop(0, NCH, body, (), unroll=False)

    return sc_gather


def _sc_gather(x, col3):
    return _build_sc_gather()(x, col3)


@functools.cache
def _build_sc_scatter():
    mesh = plsc.VectorSubcoreMesh(core_axis_name="c", subcore_axis_name="s")

    @functools.partial(
        pl.kernel,
        out_type=jax.ShapeDtypeStruct((2, NP_PAD, H), jnp.float32),
        mesh=mesh,
        compiler_params=pltpu.CompilerParams(use_tc_tiling_on_sc=False),
        scratch_types=[
            pltpu.VMEM((NCH, CH), jnp.int32),
            pltpu.VMEM((CH, H), jnp.float32),
            pltpu.VMEM_SHARED((NP_PAD, H), jnp.float32),
            pltpu.SemaphoreType.DMA,
        ],
    )
    def sc_scatter(msg_hbm, row_hbm, zero_hbm, out_hbm, idx_v, rows_v, acc_sh, sem):
        c = lax.axis_index("c")
        s = lax.axis_index("s")
        wid = s * 2 + c
        pltpu.sync_copy(
            zero_hbm.at[pl.ds(s * NPT, NPT)], acc_sh.at[pl.ds(s * NPT, NPT)]
        )
        plsc.subcore_barrier()
        pltpu.sync_copy(row_hbm.at[wid], idx_v)

        def body(j, _):
            pltpu.sync_copy(msg_hbm.at[pl.ds(wid * EPW + j * CH, CH)], rows_v)
            pltpu.sync_copy(rows_v, acc_sh.at[idx_v.at[j]], add=True)
            return ()

        lax.fori_loop(0, NCH, body, (), unroll=False)
        plsc.subcore_barrier()
        pltpu.sync_copy(
            acc_sh.at[pl.ds(s * NPT, NPT)], out_hbm.at[c].at[pl.ds(s * NPT, NPT)]
        )

    return sc_scatter


def _sc_scatter(msg, row3, zeros):
    return _build_sc_scatter()(msg, row3, zeros)


BE = 2000


def _msg_body(xs_ref, y_ref, w1_ref, wl_ref, out_ref):
    xs = xs_ref[...]
    acc = jnp.zeros((BE, H), jnp.float32)
    for j in range(DE):
        zj = jnp.dot(xs, w1_ref[j], preferred_element_type=jnp.float32)
        acc = acc + zj * y_ref[:, j : j + 1]
    acc = acc * (1.0 / math.sqrt(D * DE))
    sig = 1.0 / (1.0 + jnp.exp(-acc))
    s = acc * sig
    out_ref[...] = jnp.dot(
        s, wl_ref[...], preferred_element_type=jnp.float32
    ) * (1.0 / math.sqrt(H))


def _msg_call(x_src, y, w1t, wl):
    return pl.pallas_call(
        _msg_body,
        grid=(E // BE,),
        in_specs=[
            pl.BlockSpec((BE, D), lambda i: (i, 0)),
            pl.BlockSpec((BE, DE), lambda i: (i, 0)),
            pl.BlockSpec((DE, D, H), lambda i: (0, 0, 0)),
            pl.BlockSpec((H, H), lambda i: (0, 0)),
        ],
        out_specs=pl.BlockSpec((BE, H), lambda i: (i, 0)),
        out_shape=jax.ShapeDtypeStruct((E, H), jnp.float32),
    )(x_src, y, w1t, wl)


BN = 400


def _upd_body(x_ref, agg_ref, w2_ref, wl_ref, out_ref):
    x = x_ref[...]
    agg = agg_ref[0] + agg_ref[1]
    c = jnp.dot(x, w2_ref[...], preferred_element_type=jnp.float32)
    c3 = c.reshape(BN, H, H)
    u = jnp.sum(c3 * agg[:, :, None], axis=1) * (1.0 / math.sqrt(D * H))
    sig = 1.0 / (1.0 + jnp.exp(-u))
    u = u * sig
    out_ref[...] = x + jnp.dot(
        u, wl_ref[...], preferred_element_type=jnp.float32
    ) * (1.0 / math.sqrt(H))


def _upd_call(x, agg2, w2flat, wl):
    return pl.pallas_call(
        _upd_body,
        grid=(N // BN,),
        in_specs=[
            pl.BlockSpec((BN, D), lambda i: (i, 0)),
            pl.BlockSpec((2, BN, H), lambda i: (0, i, 0)),
            pl.BlockSpec((D, H * H), lambda i: (0, 0)),
            pl.BlockSpec((H, D), lambda i: (0, 0)),
        ],
        out_specs=pl.BlockSpec((BN, D), lambda i: (i, 0)),
        out_shape=jax.ShapeDtypeStruct((N, D), jnp.float32),
    )(x, agg2, w2flat, wl)


def kernel(node_features, edge_index, edge_attr_e3nn, node_attr_scalar_raw,
           W_tp_msg, W_lin_msg, W_tp_upd, W_lin_upd):
    del node_attr_scalar_raw
    row = edge_index[0].reshape(NW, NCH, CH)
    col = edge_index[1].reshape(NW, NCH, CH)
    x_src = _sc_gather(node_features, col)
    msg = _msg_call(x_src, edge_attr_e3nn, W_tp_msg.transpose(1, 0, 2), W_lin_msg)
    zeros = jnp.zeros((NP_PAD, H), jnp.float32)
    agg2 = _sc_scatter(msg, row, zeros)
    return _upd_call(node_features, agg2, W_tp_upd.reshape(D, H * H), W_lin_upd)


# speedup vs baseline: 1.8207x; 1.1936x over previous
import functools
import math

import jax
import jax.numpy as jnp
from jax import lax
from jax.experimental import pallas as pl
from jax.experimental.pallas import tpu as pltpu
from jax.experimental.pallas import tpu_sc as plsc

N, E, D, DE, H = 10000, 160000, 128, 4, 64

NW = 32
EPW = E // NW
CH = 40
NCH = EPW // CH
NP_PAD = 10240
NPT = NP_PAD // 16

@functools.cache
def _build_sc_gather():
    mesh = plsc.VectorSubcoreMesh(core_axis_name="c", subcore_axis_name="s")

    @functools.partial(
        pl.kernel,
        out_type=jax.ShapeDtypeStruct((E, D), jnp.float32),
        mesh=mesh,
        scratch_types=[
            pltpu.VMEM((NCH, CH), jnp.int32),
            pltpu.VMEM((CH, D), jnp.float32),
            pltpu.SemaphoreType.DMA,
        ],
    )
    def sc_gather(x_hbm, col_hbm, out_hbm, idx_v, rows_v, sem):
        wid = lax.axis_index("s") * 2 + lax.axis_index("c")
        pltpu.sync_copy(col_hbm.at[wid], idx_v)

        def body(j, _):
            pltpu.async_copy(x_hbm.at[idx_v.at[j]], rows_v, sem).wait()
            pltpu.sync_copy(rows_v, out_hbm.at[pl.ds(wid * EPW + j * CH, CH)])
            return ()

        lax.fori_loop(0, NCH, body, (), unroll=False)

    return sc_gather


def _sc_gather(x, col3):
    return _build_sc_gather()(x, col3)


@functools.cache
def _build_sc_scatter():
    mesh = plsc.VectorSubcoreMesh(core_axis_name="c", subcore_axis_name="s")

    @functools.partial(
        pl.kernel,
        out_type=jax.ShapeDtypeStruct((2, NP_PAD, H), jnp.float32),
        mesh=mesh,
        compiler_params=pltpu.CompilerParams(use_tc_tiling_on_sc=False),
        scratch_types=[
            pltpu.VMEM((NCH, CH), jnp.int32),
            pltpu.VMEM((CH, H), jnp.float32),
            pltpu.VMEM_SHARED((NP_PAD, H), jnp.float32),
            pltpu.SemaphoreType.DMA,
        ],
    )
    def sc_scatter(msg_hbm, row_hbm, zero_hbm, out_hbm, idx_v, rows_v, acc_sh, sem):
        c = lax.axis_index("c")
        s = lax.axis_index("s")
        wid = s * 2 + c
        pltpu.sync_copy(
            zero_hbm.at[pl.ds(s * NPT, NPT)], acc_sh.at[pl.ds(s * NPT, NPT)]
        )
        plsc.subcore_barrier()
        pltpu.sync_copy(row_hbm.at[wid], idx_v)

        def body(j, _):
            pltpu.sync_copy(msg_hbm.at[pl.ds(wid * EPW + j * CH, CH)], rows_v)
            pltpu.sync_copy(rows_v, acc_sh.at[idx_v.at[j]], add=True)
            return ()

        lax.fori_loop(0, NCH, body, (), unroll=False)
        plsc.subcore_barrier()
        pltpu.sync_copy(
            acc_sh.at[pl.ds(s * NPT, NPT)], out_hbm.at[c].at[pl.ds(s * NPT, NPT)]
        )

    return sc_scatter


def _sc_scatter(msg, row3, zeros):
    return _build_sc_scatter()(msg, row3, zeros)


BE = 4000


def _msg_body(xs_ref, y_ref, w1_ref, r4_ref, s4_ref, wl_ref, out_ref):
    xs = xs_ref[...].astype(jnp.bfloat16)
    z = jnp.dot(xs, w1_ref[...], preferred_element_type=jnp.float32)
    yx = jnp.dot(y_ref[...], r4_ref[...], preferred_element_type=jnp.float32)
    m = (z * yx).astype(jnp.bfloat16)
    pre = jnp.dot(m, s4_ref[...], preferred_element_type=jnp.float32) * (
        1.0 / math.sqrt(D * DE)
    )
    sig = 1.0 / (1.0 + jnp.exp(-pre))
    s = (pre * sig).astype(jnp.bfloat16)
    out_ref[...] = jnp.dot(
        s, wl_ref[...], preferred_element_type=jnp.float32
    ) * (1.0 / math.sqrt(H))


def _msg_call(x_src, y, w1cat, r4, s4, wl):
    return pl.pallas_call(
        _msg_body,
        grid=(E // BE,),
        in_specs=[
            pl.BlockSpec((BE, D), lambda i: (i, 0)),
            pl.BlockSpec((BE, DE), lambda i: (i, 0)),
            pl.BlockSpec((D, DE * H), lambda i: (0, 0)),
            pl.BlockSpec((DE, DE * H), lambda i: (0, 0)),
            pl.BlockSpec((DE * H, H), lambda i: (0, 0)),
            pl.BlockSpec((H, H), lambda i: (0, 0)),
        ],
        out_specs=pl.BlockSpec((BE, H), lambda i: (i, 0)),
        out_shape=jax.ShapeDtypeStruct((E, H), jnp.float32),
    )(x_src, y, w1cat, r4, s4, wl)


BN = 1000


def _upd_body(x_ref, agg_ref, w2_ref, r64_ref, s64_ref, wl_ref, out_ref):
    x = x_ref[...]
    agg = (agg_ref[0] + agg_ref[1]).astype(jnp.bfloat16)
    c = jnp.dot(x.astype(jnp.bfloat16), w2_ref[...], preferred_element_type=jnp.float32)
    agg_exp = jnp.dot(agg, r64_ref[...], preferred_element_type=jnp.float32)
    m = (c * agg_exp).astype(jnp.bfloat16)
    u = jnp.dot(m, s64_ref[...], preferred_element_type=jnp.float32) * (
        1.0 / math.sqrt(D * H)
    )
    sig = 1.0 / (1.0 + jnp.exp(-u))
    u = (u * sig).astype(jnp.bfloat16)
    out_ref[...] = x + jnp.dot(
        u, wl_ref[...], preferred_element_type=jnp.float32
    ) * (1.0 / math.sqrt(H))


def _upd_call(x, agg2, w2flat, r64, s64, wl):
    return pl.pallas_call(
        _upd_body,
        grid=(N // BN,),
        in_specs=[
            pl.BlockSpec((BN, D), lambda i: (i, 0)),
            pl.BlockSpec((2, BN, H), lambda i: (0, i, 0)),
            pl.BlockSpec((D, H * H), lambda i: (0, 0)),
            pl.BlockSpec((H, H * H), lambda i: (0, 0)),
            pl.BlockSpec((H * H, H), lambda i: (0, 0)),
            pl.BlockSpec((H, D), lambda i: (0, 0)),
        ],
        out_specs=pl.BlockSpec((BN, D), lambda i: (i, 0)),
        out_shape=jax.ShapeDtypeStruct((N, D), jnp.float32),
    )(x, agg2, w2flat, r64, s64, wl)


def kernel(node_features, edge_index, edge_attr_e3nn, node_attr_scalar_raw,
           W_tp_msg, W_lin_msg, W_tp_upd, W_lin_upd):
    del node_attr_scalar_raw
    row = edge_index[0].reshape(NW, NCH, CH)
    col = edge_index[1].reshape(NW, NCH, CH)
    gid4 = jnp.arange(DE * H, dtype=jnp.int32) // H
    r4 = (gid4[None, :] == jnp.arange(DE, dtype=jnp.int32)[:, None]).astype(jnp.float32)
    hid4 = jnp.arange(DE * H, dtype=jnp.int32) % H
    s4 = (hid4[:, None] == jnp.arange(H, dtype=jnp.int32)[None, :]).astype(jnp.float32)
    gid64 = jnp.arange(H * H, dtype=jnp.int32) // H
    r64 = (gid64[None, :] == jnp.arange(H, dtype=jnp.int32)[:, None]).astype(jnp.float32)
    hid64 = jnp.arange(H * H, dtype=jnp.int32) % H
    s64 = (hid64[:, None] == jnp.arange(H, dtype=jnp.int32)[None, :]).astype(jnp.float32)
    bf = jnp.bfloat16
    w1cat = W_tp_msg.reshape(D, DE * H).astype(bf)
    x_src = _sc_gather(node_features, col)
    msg = _msg_call(x_src, edge_attr_e3nn.astype(bf), w1cat, r4.astype(bf),
                    s4.astype(bf), W_lin_msg.astype(bf))
    zeros = jnp.zeros((NP_PAD, H), jnp.float32)
    agg2 = _sc_scatter(msg, row, zeros)
    return _upd_call(node_features, agg2, W_tp_upd.reshape(D, H * H).astype(bf),
                     r64.astype(bf), s64.astype(bf), W_lin_upd.astype(bf))


# speedup vs baseline: 2.5198x; 1.3840x over previous
import functools
import math

import jax
import jax.numpy as jnp
from jax import lax
from jax.experimental import pallas as pl
from jax.experimental.pallas import tpu as pltpu
from jax.experimental.pallas import tpu_sc as plsc

N, E, D, DE, H = 10000, 160000, 128, 4, 64

NW = 32
EPW = E // NW
CH = 40
NCH = EPW // CH
NP_PAD = 10240
NPT = NP_PAD // 16

NBUF = 5
NGRP = NCH // NBUF


@functools.cache
def _build_sc_gather():
    mesh = plsc.VectorSubcoreMesh(core_axis_name="c", subcore_axis_name="s")

    @functools.partial(
        pl.kernel,
        out_type=jax.ShapeDtypeStruct((E, D), jnp.float32),
        mesh=mesh,
        scratch_types=[
            pltpu.VMEM((NCH, CH), jnp.int32),
            pltpu.VMEM((NBUF, CH, D), jnp.float32),
        ]
        + [pltpu.SemaphoreType.DMA] * (2 * NBUF),
    )
    def sc_gather(x_hbm, col_hbm, out_hbm, idx_v, rows_v, *sems):
        sgs, sws = sems[:NBUF], sems[NBUF:]
        wid = lax.axis_index("s") * 2 + lax.axis_index("c")
        base = wid * EPW
        pltpu.sync_copy(col_hbm.at[wid], idx_v)

        def g_desc(j, b):
            return pltpu.make_async_copy(x_hbm.at[idx_v.at[j]], rows_v.at[b], sgs[b])

        def w_desc(j, b):
            return pltpu.make_async_copy(
                rows_v.at[b], out_hbm.at[pl.ds(base + j * CH, CH)], sws[b]
            )

        for b in range(NBUF):
            g_desc(b, b).start()

        def body(g, _):
            j0 = g * NBUF
            for b in range(NBUF):
                g_desc(j0 + b, b).wait()
                w_desc(j0 + b, b).start()
            for b in range(NBUF):
                w_desc(j0 + b, b).wait()
                g_desc(j0 + NBUF + b, b).start()
            return ()

        lax.fori_loop(0, NGRP - 1, body, (), unroll=False)
        j0 = (NGRP - 1) * NBUF
        for b in range(NBUF):
            g_desc(j0 + b, b).wait()
            w_desc(j0 + b, b).start()
        for b in range(NBUF):
            w_desc(j0 + b, b).wait()

    return sc_gather


def _sc_gather(x, col3):
    return _build_sc_gather()(x, col3)


@functools.cache
def _build_sc_scatter():
    mesh = plsc.VectorSubcoreMesh(core_axis_name="c", subcore_axis_name="s")

    @functools.partial(
        pl.kernel,
        out_type=jax.ShapeDtypeStruct((2, NP_PAD, H), jnp.float32),
        mesh=mesh,
        compiler_params=pltpu.CompilerParams(use_tc_tiling_on_sc=False),
        scratch_types=[
            pltpu.VMEM((NCH, CH), jnp.int32),
            pltpu.VMEM((NBUF, CH, H), jnp.float32),
            pltpu.VMEM_SHARED((NP_PAD, H), jnp.float32),
        ]
        + [pltpu.SemaphoreType.DMA] * (2 * NBUF),
    )
    def sc_scatter(msg_hbm, row_hbm, zero_hbm, out_hbm, idx_v, rows_v, acc_sh, *sems):
        sls, sss = sems[:NBUF], sems[NBUF:]
        c = lax.axis_index("c")
        s = lax.axis_index("s")
        wid = s * 2 + c
        base = wid * EPW
        pltpu.sync_copy(
            zero_hbm.at[pl.ds(s * NPT, NPT)], acc_sh.at[pl.ds(s * NPT, NPT)]
        )
        plsc.subcore_barrier()
        pltpu.sync_copy(row_hbm.at[wid], idx_v)

        def l_desc(j, b):
            return pltpu.make_async_copy(
                msg_hbm.at[pl.ds(base + j * CH, CH)], rows_v.at[b], sls[b]
            )

        def s_desc(j, b):
            return pltpu.make_async_copy(rows_v.at[b], acc_sh.at[idx_v.at[j]], sss[b])

        for b in range(NBUF):
            l_desc(b, b).start()

        def body(g, _):
            j0 = g * NBUF
            for b in range(NBUF):
                l_desc(j0 + b, b).wait()
                s_desc(j0 + b, b).start(add=True)
            for b in range(NBUF):
                s_desc(j0 + b, b).wait()
                l_desc(j0 + NBUF + b, b).start()
            return ()

        lax.fori_loop(0, NGRP - 1, body, (), unroll=False)
        j0 = (NGRP - 1) * NBUF
        for b in range(NBUF):
            l_desc(j0 + b, b).wait()
            s_desc(j0 + b, b).start(add=True)
        for b in range(NBUF):
            s_desc(j0 + b, b).wait()
        plsc.subcore_barrier()
        pltpu.sync_copy(
            acc_sh.at[pl.ds(s * NPT, NPT)], out_hbm.at[c].at[pl.ds(s * NPT, NPT)]
        )

    return sc_scatter


def _sc_scatter(msg, row3, zeros):
    return _build_sc_scatter()(msg, row3, zeros)


BE = 4000


def _msg_body(xs_ref, y_ref, w1_ref, r4_ref, s4_ref, wl_ref, out_ref):
    xs = xs_ref[...].astype(jnp.bfloat16)
    z = jnp.dot(xs, w1_ref[...], preferred_element_type=jnp.float32)
    yx = jnp.dot(y_ref[...], r4_ref[...], preferred_element_type=jnp.float32)
    m = (z * yx).astype(jnp.bfloat16)
    pre = jnp.dot(m, s4_ref[...], preferred_element_type=jnp.float32) * (
        1.0 / math.sqrt(D * DE)
    )
    sig = 1.0 / (1.0 + jnp.exp(-pre))
    s = (pre * sig).astype(jnp.bfloat16)
    out_ref[...] = jnp.dot(
        s, wl_ref[...], preferred_element_type=jnp.float32
    ) * (1.0 / math.sqrt(H))


def _msg_call(x_src, y, w1cat, r4, s4, wl):
    return pl.pallas_call(
        _msg_body,
        grid=(E // BE,),
        in_specs=[
            pl.BlockSpec((BE, D), lambda i: (i, 0)),
            pl.BlockSpec((BE, DE), lambda i: (i, 0)),
            pl.BlockSpec((D, DE * H), lambda i: (0, 0)),
            pl.BlockSpec((DE, DE * H), lambda i: (0, 0)),
            pl.BlockSpec((DE * H, H), lambda i: (0, 0)),
            pl.BlockSpec((H, H), lambda i: (0, 0)),
        ],
        out_specs=pl.BlockSpec((BE, H), lambda i: (i, 0)),
        out_shape=jax.ShapeDtypeStruct((E, H), jnp.float32),
    )(x_src, y, w1cat, r4, s4, wl)


BN = 1000


def _upd_body(x_ref, agg_ref, w2_ref, r64_ref, s64_ref, wl_ref, out_ref):
    x = x_ref[...]
    agg = (agg_ref[0] + agg_ref[1]).astype(jnp.bfloat16)
    c = jnp.dot(x.astype(jnp.bfloat16), w2_ref[...], preferred_element_type=jnp.float32)
    agg_exp = jnp.dot(agg, r64_ref[...], preferred_element_type=jnp.float32)
    m = (c * agg_exp).astype(jnp.bfloat16)
    u = jnp.dot(m, s64_ref[...], preferred_element_type=jnp.float32) * (
        1.0 / math.sqrt(D * H)
    )
    sig = 1.0 / (1.0 + jnp.exp(-u))
    u = (u * sig).astype(jnp.bfloat16)
    out_ref[...] = x + jnp.dot(
        u, wl_ref[...], preferred_element_type=jnp.float32
    ) * (1.0 / math.sqrt(H))


def _upd_call(x, agg2, w2flat, r64, s64, wl):
    return pl.pallas_call(
        _upd_body,
        grid=(N // BN,),
        in_specs=[
            pl.BlockSpec((BN, D), lambda i: (i, 0)),
            pl.BlockSpec((2, BN, H), lambda i: (0, i, 0)),
            pl.BlockSpec((D, H * H), lambda i: (0, 0)),
            pl.BlockSpec((H, H * H), lambda i: (0, 0)),
            pl.BlockSpec((H * H, H), lambda i: (0, 0)),
            pl.BlockSpec((H, D), lambda i: (0, 0)),
        ],
        out_specs=pl.BlockSpec((BN, D), lambda i: (i, 0)),
        out_shape=jax.ShapeDtypeStruct((N, D), jnp.float32),
    )(x, agg2, w2flat, r64, s64, wl)


def kernel(node_features, edge_index, edge_attr_e3nn, node_attr_scalar_raw,
           W_tp_msg, W_lin_msg, W_tp_upd, W_lin_upd):
    del node_attr_scalar_raw
    row = edge_index[0].reshape(NW, NCH, CH)
    col = edge_index[1].reshape(NW, NCH, CH)
    gid4 = jnp.arange(DE * H, dtype=jnp.int32) // H
    r4 = (gid4[None, :] == jnp.arange(DE, dtype=jnp.int32)[:, None]).astype(jnp.float32)
    hid4 = jnp.arange(DE * H, dtype=jnp.int32) % H
    s4 = (hid4[:, None] == jnp.arange(H, dtype=jnp.int32)[None, :]).astype(jnp.float32)
    gid64 = jnp.arange(H * H, dtype=jnp.int32) // H
    r64 = (gid64[None, :] == jnp.arange(H, dtype=jnp.int32)[:, None]).astype(jnp.float32)
    hid64 = jnp.arange(H * H, dtype=jnp.int32) % H
    s64 = (hid64[:, None] == jnp.arange(H, dtype=jnp.int32)[None, :]).astype(jnp.float32)
    bf = jnp.bfloat16
    w1cat = W_tp_msg.reshape(D, DE * H).astype(bf)
    x_src = _sc_gather(node_features, col)
    msg = _msg_call(x_src, edge_attr_e3nn.astype(bf), w1cat, r4.astype(bf),
                    s4.astype(bf), W_lin_msg.astype(bf))
    zeros = jnp.zeros((NP_PAD, H), jnp.float32)
    agg2 = _sc_scatter(msg, row, zeros)
    return _upd_call(node_features, agg2, W_tp_upd.reshape(D, H * H).astype(bf),
                     r64.astype(bf), s64.astype(bf), W_lin_upd.astype(bf))
